# Initial kernel scaffold; baseline (speedup 1.0000x reference)
import jax
import jax.numpy as jnp
from jax.experimental import pallas as pl


def kernel(z_e, codebook_pca, W, b):
    raise NotImplementedError("write your pallas kernel here")


# speedup vs baseline: 15.7483x; 15.7483x over previous
import functools

import jax
import jax.numpy as jnp
from jax import lax
from jax.experimental import pallas as pl
from jax.experimental.pallas import tpu as pltpu

_WN = 10
_K = 8192
_PCA = 4096
_D = 512
_N = 10240
_G = _N // _WN

_KB = 512
_GB = 16


def _mapped_body(cb_ref, w_ref, b_ref, mapped_ref, msq_ref):
    m = lax.dot_general(cb_ref[...], w_ref[...], (((1,), (1,)), ((), ())),
                        preferred_element_type=jnp.float32)
    m = m + b_ref[...]
    mapped_ref[...] = m
    msq_ref[...] = jnp.sum(m * m, axis=1, keepdims=True)


def _mapped_call(cb, w, b2):
    return pl.pallas_call(
        _mapped_body,
        grid=(_K // _KB,),
        in_specs=[
            pl.BlockSpec((_KB, _PCA), lambda j: (j, 0)),
            pl.BlockSpec((_D, _PCA), lambda j: (0, 0)),
            pl.BlockSpec((1, _D), lambda j: (0, 0)),
        ],
        out_specs=[
            pl.BlockSpec((_KB, _D), lambda j: (j, 0)),
            pl.BlockSpec((_KB, 1), lambda j: (j, 0)),
        ],
        out_shape=[
            jax.ShapeDtypeStruct((_K, _D), jnp.float32),
            jax.ShapeDtypeStruct((_K, 1), jnp.float32),
        ],
    )(cb, w, b2)


def _assign_body(zw_ref, mt_ref, msq_ref, idx_ref):
    zb = zw_ref[...]
    z2 = zb.reshape(_WN * _GB, _D)
    dot = lax.dot_general(z2, mt_ref[...], (((1,), (0,)), ((), ())),
                          preferred_element_type=jnp.float32)
    dot3 = dot.reshape(_WN, _GB, _K)
    zsq = jnp.sum(zb * zb, axis=-1)
    sums = zsq[:, :, None] + msq_ref[...].reshape(1, 1, _K)
    dist = sums - 2.0 * dot3
    col = lax.broadcasted_iota(jnp.int32, (_GB, _K), 1)
    masked = jnp.zeros((_GB, _K), jnp.bool_)
    cols = []
    for i in range(_WN):
        di = jnp.where(masked, jnp.inf, dist[i])
        mval = jnp.min(di, axis=1, keepdims=True)
        cand = jnp.where(di == mval, col, jnp.int32(_K))
        idx_i = jnp.min(cand, axis=1)
        cols.append(idx_i)
        masked = jnp.logical_or(masked, col == idx_i[:, None])
    idx_ref[...] = jnp.stack(cols, axis=1)


def _assign_call(zw, mt, msq_row):
    return pl.pallas_call(
        _assign_body,
        grid=(_G // _GB,),
        in_specs=[
            pl.BlockSpec((_WN, _GB, _D), lambda j: (0, j, 0)),
            pl.BlockSpec((_D, _K), lambda j: (0, 0)),
            pl.BlockSpec((1, _K), lambda j: (0, 0)),
        ],
        out_specs=pl.BlockSpec((_GB, _WN), lambda j: (j, 0)),
        out_shape=jax.ShapeDtypeStruct((_G, _WN), jnp.int32),
    )(zw, mt, msq_row)


def kernel(z_e, codebook_pca, W, b):
    mapped, msq_col = _mapped_call(codebook_pca, W, b[None, :])
    mt = mapped.T
    msq_row = msq_col.reshape(1, _K)
    zw = z_e.reshape(_G, _WN, _D).transpose(1, 0, 2)
    idxs = _assign_call(zw, mt, msq_row)
    min_idx = idxs.reshape(-1)
    z_q = mapped[min_idx]
    z_q_st = z_e + lax.stop_gradient(z_q - z_e)
    vq = jnp.mean((z_q - z_e) ** 2)
    loss = 0.75 * vq + 0.25 * vq
    return (z_q_st, loss)
